# baseline (speedup 1.0000x reference)
import functools

import jax
import jax.numpy as jnp
from jax import lax
from jax.experimental import pallas as pl
from jax.experimental.pallas import tpu as pltpu
from jax.experimental.pallas import tpu_sc as plsc

N = 50000
K = 8
D = 2
S = 2500
SP = 2560
ES = 16384
TI = 256

_F32 = jnp.float32
_HIGH = lax.Precision.HIGHEST


def _softmax0(x):
    m = jnp.max(x, axis=0, keepdims=True)
    e = jnp.exp(x - m)
    return e / jnp.sum(e, axis=0, keepdims=True)


def _tc_body(z_ref, gt_ref, a_ref, s1t_ref, s2t_ref, eit_ref, ejt_ref,
             out_ref, xb_ref):
    zs_full = _softmax0(z_ref[...])
    zg = zs_full * jax.nn.sigmoid(gt_ref[...])
    denom = jnp.sum(zg, axis=1, keepdims=True)

    s1t = s1t_ref[...]
    zs_l = _softmax0(s1t[0:K, :])
    beta_l = s1t[K:K + 1, :]
    g_l = jax.nn.sigmoid(s2t_ref[...])
    cid = lax.broadcasted_iota(jnp.int32, (1, SP), 1)
    col_valid = cid < S
    c_l = jnp.where(col_valid, zs_l * g_l / denom, 0.0)

    b_kk = lax.dot_general(zs_l, c_l, (((1,), (1,)), ((), ())),
                           preferred_element_type=_F32, precision=_HIGH)
    azc = lax.dot_general(a_ref[...], b_kk, (((1,), (0,)), ((), ())),
                          preferred_element_type=_F32, precision=_HIGH)
    x_l = lax.dot_general(azc, zs_l, (((1,), (0,)), ((), ())),
                          preferred_element_type=_F32, precision=_HIGH)

    xbt = jnp.concatenate([x_l, beta_l], axis=0)
    xb_ref[...] = xbt.T
    x0l = x_l[0:1, :]
    x1l = x_l[1:2, :]

    def body(t, acc):
        i0 = t * TI
        tile = xb_ref[pl.ds(i0, TI), :]
        xi0 = tile[:, 0:1]
        xi1 = tile[:, 1:2]
        bi = tile[:, 2:3]
        rid = i0 + lax.broadcasted_iota(jnp.int32, (TI, 1), 0)
        d0 = xi0 - x0l + 1e-6
        d1 = xi1 - x1l + 1e-6
        dist = jnp.sqrt(d0 * d0 + d1 * d1)
        m = jnp.exp(bi + beta_l - dist)
        mask = (rid != cid) & (rid < S) & col_valid
        return acc + jnp.sum(jnp.where(mask, m, 0.0))

    tot = lax.fori_loop(0, SP // TI, body, _F32(0.0))
    e1 = jnp.exp(_F32(1.0))
    z1 = 0.5 * e1 * e1 * tot

    eit = eit_ref[...]
    ejt = ejt_ref[...]
    zi = _softmax0(eit[0:K, :])
    zj = _softmax0(ejt[0:K, :])
    pi = lax.dot_general(azc, zi, (((1,), (0,)), ((), ())),
                         preferred_element_type=_F32, precision=_HIGH)
    pj = lax.dot_general(azc, zj, (((1,), (0,)), ((), ())),
                         preferred_element_type=_F32, precision=_HIGH)
    df = pi - pj + 1e-6
    nrm = jnp.sqrt(df[0:1, :] ** 2 + df[1:2, :] ** 2)
    z2 = jnp.sum(eit[K:K + 1, :] + ejt[K:K + 1, :] - nrm)

    out_ref[...] = (z2 - z1)[None, None]


_NW = 32
_SROWS = SP // _NW
_EROWS = ES // _NW


@functools.partial(
    pl.kernel,
    mesh=plsc.VectorSubcoreMesh(core_axis_name="c", subcore_axis_name="s"),
    compiler_params=pltpu.CompilerParams(use_tc_tiling_on_sc=False,
                                         needs_layout_passes=False),
    out_type=[
        jax.ShapeDtypeStruct((ES, 16), _F32),
        jax.ShapeDtypeStruct((ES, 16), _F32),
    ],
    scratch_types=[
        pltpu.VMEM((SP,), jnp.int32),
        pltpu.VMEM((_EROWS,), jnp.int32),
        pltpu.VMEM((_EROWS,), jnp.int32),
        pltpu.VMEM((_EROWS, 16), _F32),
        pltpu.SemaphoreType.DMA,
    ],
)
def _sc_edge(tbl_hbm, sidx_hbm, si_hbm, sj_hbm, ei_out, ej_out,
             sidx_v, eidx_v, pos_v, erows_v, sem):
    wid = lax.axis_index("s") * 2 + lax.axis_index("c")
    ebase = wid * _EROWS
    pltpu.sync_copy(sidx_hbm, sidx_v)

    def step(_, lohiq):
        lo, hi, q = lohiq
        mid = lax.shift_right_logical(lo + hi, 1)
        v = plsc.load_gather(sidx_v, [mid])
        le = v <= q
        lo = jnp.where(le, mid + 1, lo)
        hi = jnp.where(le, hi, mid)
        return lo, hi, q

    for ids_hbm, out in ((si_hbm, ei_out), (sj_hbm, ej_out)):
        pltpu.sync_copy(ids_hbm.at[pl.ds(ebase, _EROWS)], eidx_v)
        for g0 in range(_EROWS // 16):
            q = eidx_v[pl.ds(g0 * 16, 16)]
            lo = jnp.zeros((16,), jnp.int32)
            hi = jnp.full((16,), SP, jnp.int32)
            lo, hi, _ = lax.fori_loop(0, 12, step, (lo, hi, q))
            pos_v[pl.ds(g0 * 16, 16)] = lo - 1
        pltpu.async_copy(tbl_hbm.at[pos_v], erows_v, sem).wait()
        pltpu.sync_copy(erows_v, out.at[pl.ds(ebase, _EROWS)])


def _tc_call(Z, gate_t, A, s1t, s2t, eit, ejt):
    return pl.pallas_call(
        _tc_body,
        out_shape=jax.ShapeDtypeStruct((1, 1), _F32),
        scratch_shapes=[pltpu.VMEM((SP, 3), _F32)],
    )(Z, gate_t, A, s1t, s2t, eit, ejt)


def kernel(beta, A, Z, Gate, sample_idx, sparse_sample_i, sparse_sample_j):
    beta = beta.astype(_F32)
    sidx = jnp.concatenate(
        [sample_idx.astype(jnp.int32), jnp.full((SP - S,), N, jnp.int32)])
    si = sparse_sample_i.astype(jnp.int32)
    sj = sparse_sample_j.astype(jnp.int32)
    gate_t = Gate.T
    zsamp_t = Z[:, sidx]
    beta_samp = beta[sidx]
    g_samp_t = gate_t[:, sidx]
    s1t = jnp.concatenate(
        [zsamp_t, beta_samp[None, :], jnp.zeros((7, SP), _F32)], axis=0)
    ei, ej = _sc_edge(s1t.T, sidx, si, sj)
    return _tc_call(Z, gate_t, A, s1t, g_samp_t, ei.T, ej.T)


# speedup vs baseline: 1.1455x; 1.1455x over previous
import functools

import jax
import jax.numpy as jnp
from jax import lax
from jax.experimental import pallas as pl
from jax.experimental.pallas import tpu as pltpu
from jax.experimental.pallas import tpu_sc as plsc

N = 50000
K = 8
D = 2
S = 2500
SP = 2560
ES = 16384
TI = 256

_F32 = jnp.float32
_HIGH = lax.Precision.HIGHEST


def _softmax0(x):
    m = jnp.max(x, axis=0, keepdims=True)
    e = jnp.exp(x - m)
    return e / jnp.sum(e, axis=0, keepdims=True)


def _tc_body(z_ref, gt_ref, a_ref, s1t_ref, s2t_ref, eit_ref, ejt_ref,
             out_ref, xb_ref):
    zs_full = _softmax0(z_ref[...])
    zg = zs_full * jax.nn.sigmoid(gt_ref[...])
    denom = jnp.sum(zg, axis=1, keepdims=True)

    s1t = s1t_ref[...]
    zs_l = _softmax0(s1t[0:K, :])
    beta_l = s1t[K:K + 1, :]
    g_l = jax.nn.sigmoid(s2t_ref[...])
    cid = lax.broadcasted_iota(jnp.int32, (1, SP), 1)
    col_valid = cid < S
    c_l = jnp.where(col_valid, zs_l * g_l / denom, 0.0)

    b_kk = lax.dot_general(zs_l, c_l, (((1,), (1,)), ((), ())),
                           preferred_element_type=_F32, precision=_HIGH)
    azc = lax.dot_general(a_ref[...], b_kk, (((1,), (0,)), ((), ())),
                          preferred_element_type=_F32, precision=_HIGH)
    x_l = lax.dot_general(azc, zs_l, (((1,), (0,)), ((), ())),
                          preferred_element_type=_F32, precision=_HIGH)

    xbt = jnp.concatenate([x_l, beta_l], axis=0)
    xb_ref[...] = xbt.T
    x0l = x_l[0:1, :]
    x1l = x_l[1:2, :]

    def body(t, acc):
        i0 = t * TI
        tile = xb_ref[pl.ds(i0, TI), :]
        xi0 = tile[:, 0:1]
        xi1 = tile[:, 1:2]
        bi = tile[:, 2:3]
        rid = i0 + lax.broadcasted_iota(jnp.int32, (TI, 1), 0)
        d0 = xi0 - x0l + 1e-6
        d1 = xi1 - x1l + 1e-6
        dist = jnp.sqrt(d0 * d0 + d1 * d1)
        m = jnp.exp(bi + beta_l - dist)
        mask = (rid != cid) & (rid < S) & col_valid
        return acc + jnp.sum(jnp.where(mask, m, 0.0))

    tot = lax.fori_loop(0, SP // TI, body, _F32(0.0))
    e1 = jnp.exp(_F32(1.0))
    z1 = 0.5 * e1 * e1 * tot

    eit = eit_ref[...]
    ejt = ejt_ref[...]
    zi = _softmax0(eit[0:K, :])
    zj = _softmax0(ejt[0:K, :])
    pi = lax.dot_general(azc, zi, (((1,), (0,)), ((), ())),
                         preferred_element_type=_F32, precision=_HIGH)
    pj = lax.dot_general(azc, zj, (((1,), (0,)), ((), ())),
                         preferred_element_type=_F32, precision=_HIGH)
    df = pi - pj + 1e-6
    nrm = jnp.sqrt(df[0:1, :] ** 2 + df[1:2, :] ** 2)
    z2 = jnp.sum(eit[K:K + 1, :] + ejt[K:K + 1, :] - nrm)

    out_ref[...] = (z2 - z1)[None, None]


_NW = 32
_SROWS = SP // _NW
_EROWS = ES // _NW


@functools.partial(
    pl.kernel,
    mesh=plsc.VectorSubcoreMesh(core_axis_name="c", subcore_axis_name="s"),
    compiler_params=pltpu.CompilerParams(use_tc_tiling_on_sc=False,
                                         needs_layout_passes=False),
    out_type=[
        jax.ShapeDtypeStruct((ES, 16), _F32),
        jax.ShapeDtypeStruct((ES, 16), _F32),
    ],
    scratch_types=[
        pltpu.VMEM((SP,), jnp.int32),
        pltpu.VMEM((_EROWS,), jnp.int32),
        pltpu.VMEM((_EROWS,), jnp.int32),
        pltpu.VMEM((_EROWS, 16), _F32),
        pltpu.SemaphoreType.DMA,
    ],
)
def _sc_edge(tbl_hbm, sidx_hbm, si_hbm, sj_hbm, ei_out, ej_out,
             sidx_v, eidx_v, pos_v, erows_v, sem):
    wid = lax.axis_index("s") * 2 + lax.axis_index("c")
    ebase = wid * _EROWS
    pltpu.sync_copy(sidx_hbm, sidx_v)

    def step(_, lohiq):
        lo, hi, q = lohiq
        mid = lax.shift_right_logical(lo + hi, 1)
        v = plsc.load_gather(sidx_v, [mid])
        le = v <= q
        lo = jnp.where(le, mid + 1, lo)
        hi = jnp.where(le, hi, mid)
        return lo, hi, q

    for ids_hbm, out in ((si_hbm, ei_out), (sj_hbm, ej_out)):
        pltpu.sync_copy(ids_hbm.at[pl.ds(ebase, _EROWS)], eidx_v)
        for g0 in range(_EROWS // 16):
            q = eidx_v[pl.ds(g0 * 16, 16)]
            lo = jnp.zeros((16,), jnp.int32)
            hi = jnp.full((16,), SP, jnp.int32)
            lo, hi, _ = lax.fori_loop(0, 12, step, (lo, hi, q))
            pos_v[pl.ds(g0 * 16, 16)] = lo - 1
        pltpu.async_copy(tbl_hbm.at[pos_v], erows_v, sem).wait()
        pltpu.sync_copy(erows_v, out.at[pl.ds(ebase, _EROWS)])


def _tc_call(Z, gate_t, A, s1t, s2t, eit, ejt):
    return pl.pallas_call(
        _tc_body,
        out_shape=jax.ShapeDtypeStruct((1, 1), _F32),
        scratch_shapes=[pltpu.VMEM((SP, 3), _F32)],
    )(Z, gate_t, A, s1t, s2t, eit, ejt)


def kernel(beta, A, Z, Gate, sample_idx, sparse_sample_i, sparse_sample_j):
    beta = beta.astype(_F32)
    sidx = jnp.concatenate(
        [sample_idx.astype(jnp.int32), jnp.full((SP - S,), N, jnp.int32)])
    si = sparse_sample_i.astype(jnp.int32)
    sj = sparse_sample_j.astype(jnp.int32)
    gate_t = Gate.T
    zall = jnp.concatenate([Z, beta[None, :], gate_t], axis=0)
    samp = zall[:, sidx]
    s1t = jnp.concatenate(
        [samp[0:K + 1, :], jnp.zeros((7, SP), _F32)], axis=0)
    g_samp_t = samp[K + 1:, :]
    ei, ej = _sc_edge(s1t.T, sidx, si, sj)
    return _tc_call(Z, gate_t, A, s1t, g_samp_t, ei.T, ej.T)


# speedup vs baseline: 1.2559x; 1.0964x over previous
import functools

import jax
import jax.numpy as jnp
from jax import lax
from jax.experimental import pallas as pl
from jax.experimental.pallas import tpu as pltpu
from jax.experimental.pallas import tpu_sc as plsc

N = 50000
K = 8
D = 2
S = 2500
SP = 2560
ES = 16384
TI = 256

_F32 = jnp.float32
_HIGH = lax.Precision.HIGHEST


def _softmax0(x):
    m = jnp.max(x, axis=0, keepdims=True)
    e = jnp.exp(x - m)
    return e / jnp.sum(e, axis=0, keepdims=True)


def _tc_body(z_ref, gt_ref, a_ref, s1t_ref, s2t_ref, eit_ref, ejt_ref,
             out_ref, xb_ref):
    zs_full = _softmax0(z_ref[...])
    zg = zs_full * jax.nn.sigmoid(gt_ref[...])
    denom = jnp.sum(zg, axis=1, keepdims=True)

    s1t = s1t_ref[...]
    zs_l = _softmax0(s1t[0:K, :])
    beta_l = s1t[K:K + 1, :]
    g_l = jax.nn.sigmoid(s2t_ref[...])
    cid = lax.broadcasted_iota(jnp.int32, (1, SP), 1)
    col_valid = cid < S
    c_l = jnp.where(col_valid, zs_l * g_l / denom, 0.0)

    b_kk = lax.dot_general(zs_l, c_l, (((1,), (1,)), ((), ())),
                           preferred_element_type=_F32, precision=_HIGH)
    azc = lax.dot_general(a_ref[...], b_kk, (((1,), (0,)), ((), ())),
                          preferred_element_type=_F32, precision=_HIGH)
    x_l = lax.dot_general(azc, zs_l, (((1,), (0,)), ((), ())),
                          preferred_element_type=_F32, precision=_HIGH)

    xbt = jnp.concatenate([x_l, beta_l], axis=0)
    xb_ref[...] = xbt.T
    x0l = x_l[0:1, :]
    x1l = x_l[1:2, :]

    def body(t, acc):
        i0 = t * TI
        tile = xb_ref[pl.ds(i0, TI), :]
        xi0 = tile[:, 0:1]
        xi1 = tile[:, 1:2]
        bi = tile[:, 2:3]
        rid = i0 + lax.broadcasted_iota(jnp.int32, (TI, 1), 0)
        d0 = xi0 - x0l + 1e-6
        d1 = xi1 - x1l + 1e-6
        dist = jnp.sqrt(d0 * d0 + d1 * d1)
        m = jnp.exp(bi + beta_l - dist)
        mask = (rid != cid) & (rid < S) & col_valid
        return acc + jnp.sum(jnp.where(mask, m, 0.0))

    tot = lax.fori_loop(0, SP // TI, body, _F32(0.0))
    e1 = jnp.exp(_F32(1.0))
    z1 = 0.5 * e1 * e1 * tot

    eit = eit_ref[...]
    ejt = ejt_ref[...]
    zi = _softmax0(eit[0:K, :])
    zj = _softmax0(ejt[0:K, :])
    pi = lax.dot_general(azc, zi, (((1,), (0,)), ((), ())),
                         preferred_element_type=_F32, precision=_HIGH)
    pj = lax.dot_general(azc, zj, (((1,), (0,)), ((), ())),
                         preferred_element_type=_F32, precision=_HIGH)
    df = pi - pj + 1e-6
    nrm = jnp.sqrt(df[0:1, :] ** 2 + df[1:2, :] ** 2)
    z2 = jnp.sum(eit[K:K + 1, :] + ejt[K:K + 1, :] - nrm)

    out_ref[...] = (z2 - z1)[None, None]


_NW = 32
_SROWS = SP // _NW
_EROWS = ES // _NW


@functools.partial(
    pl.kernel,
    mesh=plsc.VectorSubcoreMesh(core_axis_name="c", subcore_axis_name="s"),
    compiler_params=pltpu.CompilerParams(use_tc_tiling_on_sc=False,
                                         needs_layout_passes=False),
    out_type=[
        jax.ShapeDtypeStruct((16, ES), _F32),
        jax.ShapeDtypeStruct((16, ES), _F32),
    ],
    scratch_types=[
        pltpu.VMEM((SP,), jnp.int32),
        pltpu.VMEM((_EROWS,), jnp.int32),
        pltpu.VMEM((_EROWS,), jnp.int32),
        pltpu.VMEM((_EROWS, 16), _F32),
        pltpu.VMEM((16, _EROWS), _F32),
        pltpu.SemaphoreType.DMA,
    ],
)
def _sc_edge(tbl_hbm, sidx_hbm, si_hbm, sj_hbm, ei_out, ej_out,
             sidx_v, eidx_v, pos_v, erows_v, et_v, sem):
    wid = lax.axis_index("s") * 2 + lax.axis_index("c")
    ebase = wid * _EROWS
    pltpu.sync_copy(sidx_hbm, sidx_v)

    def step(_, lohiq):
        lo, hi, q = lohiq
        mid = lax.shift_right_logical(lo + hi, 1)
        v = plsc.load_gather(sidx_v, [mid])
        le = v <= q
        lo = jnp.where(le, mid + 1, lo)
        hi = jnp.where(le, hi, mid)
        return lo, hi, q

    iota = lax.broadcasted_iota(jnp.int32, (16,), 0)
    for ids_hbm, out in ((si_hbm, ei_out), (sj_hbm, ej_out)):
        pltpu.sync_copy(ids_hbm.at[pl.ds(ebase, _EROWS)], eidx_v)
        for g0 in range(_EROWS // 16):
            q = eidx_v[pl.ds(g0 * 16, 16)]
            lo = jnp.zeros((16,), jnp.int32)
            hi = jnp.full((16,), SP, jnp.int32)
            lo, hi, _ = lax.fori_loop(0, 12, step, (lo, hi, q))
            pos_v[pl.ds(g0 * 16, 16)] = lo - 1
        pltpu.async_copy(tbl_hbm.at[pos_v], erows_v, sem).wait()
        for g0 in range(_EROWS // 16):
            rows = g0 * 16 + iota
            for r in range(9):
                et_v[r, pl.ds(g0 * 16, 16)] = plsc.load_gather(
                    erows_v, [rows, jnp.full((16,), r, jnp.int32)])
        pltpu.sync_copy(et_v, out.at[:, pl.ds(ebase, _EROWS)])


def _tc_call(Z, gate_t, A, s1t, s2t, eit, ejt):
    return pl.pallas_call(
        _tc_body,
        out_shape=jax.ShapeDtypeStruct((1, 1), _F32),
        scratch_shapes=[pltpu.VMEM((SP, 3), _F32)],
    )(Z, gate_t, A, s1t, s2t, eit, ejt)


def kernel(beta, A, Z, Gate, sample_idx, sparse_sample_i, sparse_sample_j):
    beta = beta.astype(_F32)
    sidx = jnp.concatenate(
        [sample_idx.astype(jnp.int32), jnp.full((SP - S,), N, jnp.int32)])
    si = sparse_sample_i.astype(jnp.int32)
    sj = sparse_sample_j.astype(jnp.int32)
    gate_t = Gate.T
    zall = jnp.concatenate([Z, beta[None, :], gate_t], axis=0)
    samp = zall[:, sidx]
    s1t = jnp.concatenate(
        [samp[0:K + 1, :], jnp.zeros((7, SP), _F32)], axis=0)
    g_samp_t = samp[K + 1:, :]
    eit, ejt = _sc_edge(s1t.T, sidx, si, sj)
    return _tc_call(Z, gate_t, A, s1t, g_samp_t, eit, ejt)


# speedup vs baseline: 1.3534x; 1.0777x over previous
import functools

import jax
import jax.numpy as jnp
from jax import lax
from jax.experimental import pallas as pl
from jax.experimental.pallas import tpu as pltpu
from jax.experimental.pallas import tpu_sc as plsc

N = 50000
K = 8
D = 2
S = 2500
SP = 2560
ES = 16384
TI = 256

_F32 = jnp.float32
_HIGH = lax.Precision.HIGHEST


def _softmax0(x):
    m = jnp.max(x, axis=0, keepdims=True)
    e = jnp.exp(x - m)
    return e / jnp.sum(e, axis=0, keepdims=True)


def _tc_body(z_ref, gt_ref, a_ref, s1t_ref, s2t_ref, eit_ref, ejt_ref,
             out_ref, xb_ref):
    zs_full = _softmax0(z_ref[...])
    zg = zs_full * jax.nn.sigmoid(gt_ref[...])
    denom = jnp.sum(zg, axis=1, keepdims=True)

    s1t = s1t_ref[...]
    zs_l = _softmax0(s1t[0:K, :])
    beta_l = s1t[K:K + 1, :]
    g_l = jax.nn.sigmoid(s2t_ref[...])
    cid = lax.broadcasted_iota(jnp.int32, (1, SP), 1)
    col_valid = cid < S
    c_l = jnp.where(col_valid, zs_l * g_l / denom, 0.0)

    b_kk = lax.dot_general(zs_l, c_l, (((1,), (1,)), ((), ())),
                           preferred_element_type=_F32, precision=_HIGH)
    azc = lax.dot_general(a_ref[...], b_kk, (((1,), (0,)), ((), ())),
                          preferred_element_type=_F32, precision=_HIGH)
    x_l = lax.dot_general(azc, zs_l, (((1,), (0,)), ((), ())),
                          preferred_element_type=_F32, precision=_HIGH)

    xbt = jnp.concatenate([x_l, beta_l], axis=0)
    xb_ref[...] = xbt.T
    x0l = x_l[0:1, :]
    x1l = x_l[1:2, :]

    def body(t, acc):
        i0 = t * TI
        tile = xb_ref[pl.ds(i0, TI), :]
        xi0 = tile[:, 0:1]
        xi1 = tile[:, 1:2]
        bi = tile[:, 2:3]
        rid = i0 + lax.broadcasted_iota(jnp.int32, (TI, 1), 0)
        d0 = xi0 - x0l + 1e-6
        d1 = xi1 - x1l + 1e-6
        dist = jnp.sqrt(d0 * d0 + d1 * d1)
        m = jnp.exp(bi + beta_l - dist)
        mask = (rid != cid) & (rid < S) & col_valid
        return acc + jnp.sum(jnp.where(mask, m, 0.0))

    tot = lax.fori_loop(0, SP // TI, body, _F32(0.0))
    e1 = jnp.exp(_F32(1.0))
    z1 = 0.5 * e1 * e1 * tot

    eit = eit_ref[...]
    ejt = ejt_ref[...]
    zi = _softmax0(eit[0:K, :])
    zj = _softmax0(ejt[0:K, :])
    pi = lax.dot_general(azc, zi, (((1,), (0,)), ((), ())),
                         preferred_element_type=_F32, precision=_HIGH)
    pj = lax.dot_general(azc, zj, (((1,), (0,)), ((), ())),
                         preferred_element_type=_F32, precision=_HIGH)
    df = pi - pj + 1e-6
    nrm = jnp.sqrt(df[0:1, :] ** 2 + df[1:2, :] ** 2)
    z2 = jnp.sum(eit[K:K + 1, :] + ejt[K:K + 1, :] - nrm)

    out_ref[...] = (z2 - z1)[None, None]


_NW = 32
_SROWS = SP // _NW
_EROWS = ES // _NW


@functools.partial(
    pl.kernel,
    mesh=plsc.VectorSubcoreMesh(core_axis_name="c", subcore_axis_name="s"),
    compiler_params=pltpu.CompilerParams(use_tc_tiling_on_sc=False,
                                         needs_layout_passes=False),
    out_type=[
        jax.ShapeDtypeStruct((16, ES), _F32),
        jax.ShapeDtypeStruct((16, ES), _F32),
    ],
    scratch_types=[
        pltpu.VMEM((SP,), jnp.int32),
        pltpu.VMEM((_EROWS,), jnp.int32),
        pltpu.VMEM((_EROWS,), jnp.int32),
        pltpu.VMEM((_EROWS,), jnp.int32),
        pltpu.VMEM((_EROWS, 16), _F32),
        pltpu.VMEM((16, _EROWS), _F32),
        pltpu.SemaphoreType.DMA,
    ],
)
def _sc_edge(tbl_hbm, sidx_hbm, si_hbm, sj_hbm, ei_out, ej_out,
             sidx_v, eidx_v, lo_v, pos_v, erows_v, et_v, sem):
    wid = lax.axis_index("s") * 2 + lax.axis_index("c")
    ebase = wid * _EROWS
    pltpu.sync_copy(sidx_hbm, sidx_v)

    ng = _EROWS // 16
    iota = lax.broadcasted_iota(jnp.int32, (16,), 0)
    for ids_hbm, out in ((si_hbm, ei_out), (sj_hbm, ej_out)):
        pltpu.sync_copy(ids_hbm.at[pl.ds(ebase, _EROWS)], eidx_v)
        for g0 in range(ng):
            lo_v[pl.ds(g0 * 16, 16)] = jnp.zeros((16,), jnp.int32)
            pos_v[pl.ds(g0 * 16, 16)] = jnp.full((16,), SP, jnp.int32)

        def stepfn(_, __):
            for g0 in range(ng):
                sl = pl.ds(g0 * 16, 16)
                lo = lo_v[sl]
                hi = pos_v[sl]
                q = eidx_v[sl]
                mid = lax.shift_right_logical(lo + hi, 1)
                le = plsc.load_gather(sidx_v, [mid]) <= q
                lo_v[sl] = jnp.where(le, mid + 1, lo)
                pos_v[sl] = jnp.where(le, hi, mid)
            return 0

        lax.fori_loop(0, 12, stepfn, 0)
        for g0 in range(ng):
            sl = pl.ds(g0 * 16, 16)
            pos_v[sl] = lo_v[sl] - 1
        pltpu.async_copy(tbl_hbm.at[pos_v], erows_v, sem).wait()
        for g0 in range(ng):
            rows = g0 * 16 + iota
            for r in range(9):
                et_v[r, pl.ds(g0 * 16, 16)] = plsc.load_gather(
                    erows_v, [rows, jnp.full((16,), r, jnp.int32)])
        pltpu.sync_copy(et_v, out.at[:, pl.ds(ebase, _EROWS)])


def _tc_call(Z, gate_t, A, s1t, s2t, eit, ejt):
    return pl.pallas_call(
        _tc_body,
        out_shape=jax.ShapeDtypeStruct((1, 1), _F32),
        scratch_shapes=[pltpu.VMEM((SP, 3), _F32)],
    )(Z, gate_t, A, s1t, s2t, eit, ejt)


def kernel(beta, A, Z, Gate, sample_idx, sparse_sample_i, sparse_sample_j):
    beta = beta.astype(_F32)
    sidx = jnp.concatenate(
        [sample_idx.astype(jnp.int32), jnp.full((SP - S,), N, jnp.int32)])
    si = sparse_sample_i.astype(jnp.int32)
    sj = sparse_sample_j.astype(jnp.int32)
    gate_t = Gate.T
    zall = jnp.concatenate([Z, beta[None, :], gate_t], axis=0)
    samp = zall[:, sidx]
    s1t = jnp.concatenate(
        [samp[0:K + 1, :], jnp.zeros((7, SP), _F32)], axis=0)
    g_samp_t = samp[K + 1:, :]
    eit, ejt = _sc_edge(s1t.T, sidx, si, sj)
    return _tc_call(Z, gate_t, A, s1t, g_samp_t, eit, ejt)


# speedup vs baseline: 1.3801x; 1.0197x over previous
import functools

import jax
import jax.numpy as jnp
from jax import lax
from jax.experimental import pallas as pl
from jax.experimental.pallas import tpu as pltpu
from jax.experimental.pallas import tpu_sc as plsc

N = 50000
K = 8
D = 2
S = 2500
SP = 2560
ES = 16384
TI = 256

_F32 = jnp.float32
_HIGH = lax.Precision.HIGHEST


def _softmax0(x):
    m = jnp.max(x, axis=0, keepdims=True)
    e = jnp.exp(x - m)
    return e / jnp.sum(e, axis=0, keepdims=True)


def _tc_denom_body(z_ref, gt_ref, out_ref):
    zs_full = _softmax0(z_ref[...])
    zg = zs_full * jax.nn.sigmoid(gt_ref[...])
    out_ref[...] = jnp.sum(zg, axis=1, keepdims=True)


def _tc_body(dn_ref, a_ref, s1t_ref, s2t_ref, eit_ref, ejt_ref,
             out_ref, xb_ref):
    denom = dn_ref[...]

    s1t = s1t_ref[...]
    zs_l = _softmax0(s1t[0:K, :])
    beta_l = s1t[K:K + 1, :]
    g_l = jax.nn.sigmoid(s2t_ref[...])
    cid = lax.broadcasted_iota(jnp.int32, (1, SP), 1)
    col_valid = cid < S
    c_l = jnp.where(col_valid, zs_l * g_l / denom, 0.0)

    b_kk = lax.dot_general(zs_l, c_l, (((1,), (1,)), ((), ())),
                           preferred_element_type=_F32, precision=_HIGH)
    azc = lax.dot_general(a_ref[...], b_kk, (((1,), (0,)), ((), ())),
                          preferred_element_type=_F32, precision=_HIGH)
    x_l = lax.dot_general(azc, zs_l, (((1,), (0,)), ((), ())),
                          preferred_element_type=_F32, precision=_HIGH)

    xbt = jnp.concatenate([x_l, beta_l], axis=0)
    xb_ref[...] = xbt.T
    x0l = x_l[0:1, :]
    x1l = x_l[1:2, :]

    def body(t, acc):
        i0 = t * TI
        tile = xb_ref[pl.ds(i0, TI), :]
        xi0 = tile[:, 0:1]
        xi1 = tile[:, 1:2]
        bi = tile[:, 2:3]
        rid = i0 + lax.broadcasted_iota(jnp.int32, (TI, 1), 0)
        d0 = xi0 - x0l + 1e-6
        d1 = xi1 - x1l + 1e-6
        dist = jnp.sqrt(d0 * d0 + d1 * d1)
        m = jnp.exp(bi + beta_l - dist)
        mask = (rid != cid) & (rid < S) & col_valid
        return acc + jnp.sum(jnp.where(mask, m, 0.0))

    tot = lax.fori_loop(0, SP // TI, body, _F32(0.0))
    e1 = jnp.exp(_F32(1.0))
    z1 = 0.5 * e1 * e1 * tot

    eit = eit_ref[...]
    ejt = ejt_ref[...]
    zi = _softmax0(eit[0:K, :])
    zj = _softmax0(ejt[0:K, :])
    pi = lax.dot_general(azc, zi, (((1,), (0,)), ((), ())),
                         preferred_element_type=_F32, precision=_HIGH)
    pj = lax.dot_general(azc, zj, (((1,), (0,)), ((), ())),
                         preferred_element_type=_F32, precision=_HIGH)
    df = pi - pj + 1e-6
    nrm = jnp.sqrt(df[0:1, :] ** 2 + df[1:2, :] ** 2)
    z2 = jnp.sum(eit[K:K + 1, :] + ejt[K:K + 1, :] - nrm)

    out_ref[...] = (z2 - z1)[None, None]


_NW = 32
_SROWS = SP // _NW
_EROWS = ES // _NW


@functools.partial(
    pl.kernel,
    mesh=plsc.VectorSubcoreMesh(core_axis_name="c", subcore_axis_name="s"),
    compiler_params=pltpu.CompilerParams(use_tc_tiling_on_sc=False,
                                         needs_layout_passes=False),
    out_type=[
        jax.ShapeDtypeStruct((16, ES), _F32),
        jax.ShapeDtypeStruct((16, ES), _F32),
    ],
    scratch_types=[
        pltpu.VMEM((SP,), jnp.int32),
        pltpu.VMEM((_EROWS,), jnp.int32),
        pltpu.VMEM((_EROWS,), jnp.int32),
        pltpu.VMEM((_EROWS,), jnp.int32),
        pltpu.VMEM((_EROWS, 16), _F32),
        pltpu.VMEM((16, _EROWS), _F32),
        pltpu.SemaphoreType.DMA,
    ],
)
def _sc_edge(tbl_hbm, sidx_hbm, si_hbm, sj_hbm, ei_out, ej_out,
             sidx_v, eidx_v, lo_v, pos_v, erows_v, et_v, sem):
    wid = lax.axis_index("s") * 2 + lax.axis_index("c")
    ebase = wid * _EROWS
    pltpu.sync_copy(sidx_hbm, sidx_v)

    ng = _EROWS // 16
    iota = lax.broadcasted_iota(jnp.int32, (16,), 0)
    for ids_hbm, out in ((si_hbm, ei_out), (sj_hbm, ej_out)):
        pltpu.sync_copy(ids_hbm.at[pl.ds(ebase, _EROWS)], eidx_v)
        for g0 in range(ng):
            lo_v[pl.ds(g0 * 16, 16)] = jnp.zeros((16,), jnp.int32)
            pos_v[pl.ds(g0 * 16, 16)] = jnp.full((16,), SP, jnp.int32)

        def stepfn(_, __):
            for g0 in range(ng):
                sl = pl.ds(g0 * 16, 16)
                lo = lo_v[sl]
                hi = pos_v[sl]
                q = eidx_v[sl]
                mid = lax.shift_right_logical(lo + hi, 1)
                le = plsc.load_gather(sidx_v, [mid]) <= q
                lo_v[sl] = jnp.where(le, mid + 1, lo)
                pos_v[sl] = jnp.where(le, hi, mid)
            return 0

        lax.fori_loop(0, 12, stepfn, 0)
        for g0 in range(ng):
            sl = pl.ds(g0 * 16, 16)
            pos_v[sl] = lo_v[sl] - 1
        pltpu.async_copy(tbl_hbm.at[pos_v], erows_v, sem).wait()
        for g0 in range(ng):
            rows = g0 * 16 + iota
            for r in range(9):
                et_v[r, pl.ds(g0 * 16, 16)] = plsc.load_gather(
                    erows_v, [rows, jnp.full((16,), r, jnp.int32)])
        pltpu.sync_copy(et_v, out.at[:, pl.ds(ebase, _EROWS)])


def _tc_call(Z, gate_t, A, s1t, s2t, eit, ejt):
    denom = pl.pallas_call(
        _tc_denom_body,
        out_shape=jax.ShapeDtypeStruct((K, 1), _F32),
    )(Z, gate_t)
    return pl.pallas_call(
        _tc_body,
        out_shape=jax.ShapeDtypeStruct((1, 1), _F32),
        scratch_shapes=[pltpu.VMEM((SP, 3), _F32)],
    )(denom, A, s1t, s2t, eit, ejt)


def kernel(beta, A, Z, Gate, sample_idx, sparse_sample_i, sparse_sample_j):
    beta = beta.astype(_F32)
    sidx = jnp.concatenate(
        [sample_idx.astype(jnp.int32), jnp.full((SP - S,), N, jnp.int32)])
    si = sparse_sample_i.astype(jnp.int32)
    sj = sparse_sample_j.astype(jnp.int32)
    gate_t = Gate.T
    zall = jnp.concatenate([Z, beta[None, :], gate_t], axis=0)
    samp = zall[:, sidx]
    s1t = jnp.concatenate(
        [samp[0:K + 1, :], jnp.zeros((7, SP), _F32)], axis=0)
    g_samp_t = samp[K + 1:, :]
    eit, ejt = _sc_edge(s1t.T, sidx, si, sj)
    return _tc_call(Z, gate_t, A, s1t, g_samp_t, eit, ejt)


# speedup vs baseline: 1.3952x; 1.0109x over previous
import functools

import jax
import jax.numpy as jnp
from jax import lax
from jax.experimental import pallas as pl
from jax.experimental.pallas import tpu as pltpu
from jax.experimental.pallas import tpu_sc as plsc

N = 50000
K = 8
D = 2
S = 2500
SP = 2560
ES = 16384
TI = 512

_F32 = jnp.float32
_HIGH = lax.Precision.HIGHEST


def _softmax0(x):
    m = jnp.max(x, axis=0, keepdims=True)
    e = jnp.exp(x - m)
    return e / jnp.sum(e, axis=0, keepdims=True)


def _tc_denom_body(z_ref, gt_ref, out_ref):
    zs_full = _softmax0(z_ref[...])
    zg = zs_full * jax.nn.sigmoid(gt_ref[...])
    out_ref[...] = jnp.sum(zg, axis=1, keepdims=True)


def _tc_body(dn_ref, a_ref, s1t_ref, s2t_ref, eit_ref, ejt_ref,
             out_ref, xb_ref):
    denom = dn_ref[...]

    s1t = s1t_ref[...]
    zs_l = _softmax0(s1t[0:K, :])
    beta_l = s1t[K:K + 1, :]
    g_l = jax.nn.sigmoid(s2t_ref[...])
    cid = lax.broadcasted_iota(jnp.int32, (1, SP), 1)
    col_valid = cid < S
    c_l = jnp.where(col_valid, zs_l * g_l / denom, 0.0)

    b_kk = lax.dot_general(zs_l, c_l, (((1,), (1,)), ((), ())),
                           preferred_element_type=_F32, precision=_HIGH)
    azc = lax.dot_general(a_ref[...], b_kk, (((1,), (0,)), ((), ())),
                          preferred_element_type=_F32, precision=_HIGH)
    x_l = lax.dot_general(azc, zs_l, (((1,), (0,)), ((), ())),
                          preferred_element_type=_F32, precision=_HIGH)

    xbt = jnp.concatenate([x_l, beta_l], axis=0)
    xb_ref[...] = xbt.T
    x0l = x_l[0:1, :]
    x1l = x_l[1:2, :]

    def body(t, acc):
        i0 = t * TI
        tile = xb_ref[pl.ds(i0, TI), :]
        xi0 = tile[:, 0:1]
        xi1 = tile[:, 1:2]
        bi = tile[:, 2:3]
        rid = i0 + lax.broadcasted_iota(jnp.int32, (TI, 1), 0)
        d0 = xi0 - x0l + 1e-6
        d1 = xi1 - x1l + 1e-6
        dist = jnp.sqrt(d0 * d0 + d1 * d1)
        m = jnp.exp(bi + beta_l - dist)
        mask = (rid != cid) & (rid < S) & col_valid
        return acc + jnp.sum(jnp.where(mask, m, 0.0))

    tot = lax.fori_loop(0, SP // TI, body, _F32(0.0))
    e1 = jnp.exp(_F32(1.0))
    z1 = 0.5 * e1 * e1 * tot

    eit = eit_ref[...]
    ejt = ejt_ref[...]
    zi = _softmax0(eit[0:K, :])
    zj = _softmax0(ejt[0:K, :])
    pi = lax.dot_general(azc, zi, (((1,), (0,)), ((), ())),
                         preferred_element_type=_F32, precision=_HIGH)
    pj = lax.dot_general(azc, zj, (((1,), (0,)), ((), ())),
                         preferred_element_type=_F32, precision=_HIGH)
    df = pi - pj + 1e-6
    nrm = jnp.sqrt(df[0:1, :] ** 2 + df[1:2, :] ** 2)
    z2 = jnp.sum(eit[K:K + 1, :] + ejt[K:K + 1, :] - nrm)

    out_ref[...] = (z2 - z1)[None, None]


_NW = 32
_SROWS = SP // _NW
_EROWS = ES // _NW


@functools.partial(
    pl.kernel,
    mesh=plsc.VectorSubcoreMesh(core_axis_name="c", subcore_axis_name="s"),
    compiler_params=pltpu.CompilerParams(use_tc_tiling_on_sc=False,
                                         needs_layout_passes=False),
    out_type=[
        jax.ShapeDtypeStruct((16, ES), _F32),
        jax.ShapeDtypeStruct((16, ES), _F32),
    ],
    scratch_types=[
        pltpu.VMEM((SP,), jnp.int32),
        pltpu.VMEM((_EROWS,), jnp.int32),
        pltpu.VMEM((_EROWS,), jnp.int32),
        pltpu.VMEM((_EROWS,), jnp.int32),
        pltpu.VMEM((_EROWS, 16), _F32),
        pltpu.VMEM((16, _EROWS), _F32),
        pltpu.SemaphoreType.DMA,
    ],
)
def _sc_edge(tbl_hbm, sidx_hbm, si_hbm, sj_hbm, ei_out, ej_out,
             sidx_v, eidx_v, lo_v, pos_v, erows_v, et_v, sem):
    wid = lax.axis_index("s") * 2 + lax.axis_index("c")
    ebase = wid * _EROWS
    pltpu.sync_copy(sidx_hbm, sidx_v)

    ng = _EROWS // 16
    iota = lax.broadcasted_iota(jnp.int32, (16,), 0)
    for ids_hbm, out in ((si_hbm, ei_out), (sj_hbm, ej_out)):
        pltpu.sync_copy(ids_hbm.at[pl.ds(ebase, _EROWS)], eidx_v)
        for g0 in range(ng):
            lo_v[pl.ds(g0 * 16, 16)] = jnp.zeros((16,), jnp.int32)
            pos_v[pl.ds(g0 * 16, 16)] = jnp.full((16,), SP, jnp.int32)

        def stepfn(_, __):
            for g0 in range(ng):
                sl = pl.ds(g0 * 16, 16)
                lo = lo_v[sl]
                hi = pos_v[sl]
                q = eidx_v[sl]
                mid = lax.shift_right_logical(lo + hi, 1)
                le = plsc.load_gather(sidx_v, [mid]) <= q
                lo_v[sl] = jnp.where(le, mid + 1, lo)
                pos_v[sl] = jnp.where(le, hi, mid)
            return 0

        lax.fori_loop(0, 12, stepfn, 0)
        for g0 in range(ng):
            sl = pl.ds(g0 * 16, 16)
            pos_v[sl] = lo_v[sl] - 1
        pltpu.async_copy(tbl_hbm.at[pos_v], erows_v, sem).wait()
        for g0 in range(ng):
            rows = g0 * 16 + iota
            for r in range(9):
                et_v[r, pl.ds(g0 * 16, 16)] = plsc.load_gather(
                    erows_v, [rows, jnp.full((16,), r, jnp.int32)])
        pltpu.sync_copy(et_v, out.at[:, pl.ds(ebase, _EROWS)])


def _tc_call(Z, gate_t, A, s1t, s2t, eit, ejt):
    denom = pl.pallas_call(
        _tc_denom_body,
        out_shape=jax.ShapeDtypeStruct((K, 1), _F32),
    )(Z, gate_t)
    return pl.pallas_call(
        _tc_body,
        out_shape=jax.ShapeDtypeStruct((1, 1), _F32),
        scratch_shapes=[pltpu.VMEM((SP, 3), _F32)],
    )(denom, A, s1t, s2t, eit, ejt)


def kernel(beta, A, Z, Gate, sample_idx, sparse_sample_i, sparse_sample_j):
    beta = beta.astype(_F32)
    sidx = jnp.concatenate(
        [sample_idx.astype(jnp.int32), jnp.full((SP - S,), N, jnp.int32)])
    si = sparse_sample_i.astype(jnp.int32)
    sj = sparse_sample_j.astype(jnp.int32)
    gate_t = Gate.T
    zall = jnp.concatenate([Z, beta[None, :], gate_t], axis=0)
    samp = zall[:, sidx]
    s1t = jnp.concatenate(
        [samp[0:K + 1, :], jnp.zeros((7, SP), _F32)], axis=0)
    g_samp_t = samp[K + 1:, :]
    eit, ejt = _sc_edge(s1t.T, sidx, si, sj)
    return _tc_call(Z, gate_t, A, s1t, g_samp_t, eit, ejt)
